# baseline (speedup 1.0000x reference)
import functools

import jax
import jax.numpy as jnp
from jax.experimental import pallas as pl
from jax.experimental.pallas import tpu as pltpu

_K = 32


def _lrelu(t):
    return jnp.where(t >= 0, t, 0.1 * t)


def _det_body(x_ref, w1_ref, b1_ref, w2_ref, b2_ref, o_ref):
    h = _lrelu(jnp.dot(x_ref[...], w1_ref[...],
                       preferred_element_type=jnp.float32) + b1_ref[...])
    logit = jnp.dot(h, w2_ref[...], preferred_element_type=jnp.float32)
    o_ref[...] = jax.nn.sigmoid(logit + b2_ref[...])


def _detector(x2d, W1, b1, W2, b2, rb):
    M, H = x2d.shape
    return pl.pallas_call(
        _det_body,
        grid=(M // rb,),
        in_specs=[
            pl.BlockSpec((rb, H), lambda i: (i, 0)),
            pl.BlockSpec((H, H), lambda i: (0, 0)),
            pl.BlockSpec((1, H), lambda i: (0, 0)),
            pl.BlockSpec((H, 1), lambda i: (0, 0)),
            pl.BlockSpec((1, 1), lambda i: (0, 0)),
        ],
        out_specs=pl.BlockSpec((rb, 1), lambda i: (i, 0)),
        out_shape=jax.ShapeDtypeStruct((M, 1), jnp.float32),
    )(x2d, W1, b1.reshape(1, H), W2, b2.reshape(1, 1))


def _topk_body(p_ref, vals_ref, idx_ref, gidx_ref):
    B, S = p_ref.shape
    p0 = p_ref[...]
    col = jax.lax.broadcasted_iota(jnp.int32, (B, S), 1)
    kcol = jax.lax.broadcasted_iota(jnp.int32, (B, _K), 1)

    def body(j, carry):
        p, vals, idxs = carry
        m = jnp.max(p, axis=1, keepdims=True)
        cand = jnp.where(p == m, col, S)
        i = jnp.min(cand, axis=1, keepdims=True)
        vals = jnp.where(kcol == j, m, vals)
        idxs = jnp.where(kcol == j, i, idxs)
        p = jnp.where(col == i, -1.0, p)
        return p, vals, idxs

    _, vals, idxs = jax.lax.fori_loop(
        0, _K, body,
        (p0, jnp.zeros((B, _K), jnp.float32), jnp.zeros((B, _K), jnp.int32)))
    vals_ref[...] = vals
    idx_ref[...] = idxs
    row = jax.lax.broadcasted_iota(jnp.int32, (B, _K), 0)
    gidx_ref[...] = idxs + row * S


def _topk(probs):
    B, S = probs.shape
    return pl.pallas_call(
        _topk_body,
        out_shape=(
            jax.ShapeDtypeStruct((B, _K), jnp.float32),
            jax.ShapeDtypeStruct((B, _K), jnp.int32),
            jax.ShapeDtypeStruct((B, _K), jnp.int32),
        ),
    )(probs)


from jax import lax
from jax.experimental.pallas import tpu_sc as plsc


def _sc_topk_gather(probs, x2d):
    B, S = probs.shape
    M, H = x2d.shape
    NCH = S // 16
    NG = NCH // 16
    mesh = plsc.VectorSubcoreMesh(core_axis_name="c", subcore_axis_name="s")

    @functools.partial(
        pl.kernel,
        mesh=mesh,
        compiler_params=pltpu.CompilerParams(needs_layout_passes=False),
        out_type=(
            jax.ShapeDtypeStruct((B, _K), jnp.float32),
            jax.ShapeDtypeStruct((B, _K), jnp.int32),
            jax.ShapeDtypeStruct((B * _K, H), jnp.float32),
        ),
        scratch_types=[
            pltpu.VMEM((S,), jnp.float32),
            pltpu.VMEM((NCH,), jnp.float32),
            pltpu.VMEM((_K,), jnp.float32),
            pltpu.VMEM((_K,), jnp.int32),
            pltpu.VMEM((_K,), jnp.int32),
            pltpu.VMEM((_K, H), jnp.float32),
            pltpu.SemaphoreType.DMA,
        ],
    )
    def sc_kernel(probs_hbm, x_hbm, tvals_hbm, tidx_hbm, xg_hbm,
                  p_v, cmax_v, vals_v, lidx_v, gidx_v, rows_v, sem):
        cid = lax.axis_index("c")
        sid = lax.axis_index("s")
        w = sid * 2 + cid

        @pl.when(w < B)
        def _():
            b = w
            lane = lax.broadcasted_iota(jnp.int32, (16,), 0)
            zi = jnp.zeros((16,), jnp.int32)
            zf = jnp.zeros((16,), jnp.float32)
            pltpu.sync_copy(probs_hbm.at[b], p_v)

            for g in range(NG):
                acc = zf - 2.0
                for l in range(16):
                    v = p_v[pl.ds((g * 16 + l) * 16, 16)]
                    acc = jnp.where(lane == l, jnp.max(v), acc)
                cmax_v[pl.ds(g * 16, 16)] = acc

            vlo = vhi = zf
            ilo = ihi = zi
            for j in range(_K):
                bestv = zf - 2.0
                bestc = zi
                for g in range(NG):
                    gv = cmax_v[pl.ds(g * 16, 16)]
                    better = gv > bestv
                    bestv = jnp.where(better, gv, bestv)
                    bestc = jnp.where(better, g * 16 + lane, bestc)
                mv = jnp.max(bestv)
                ch = jnp.min(jnp.where(bestv == mv, bestc, NCH))
                cidx = ch * 16 + lane
                chunk = plsc.load_gather(p_v, [cidx])
                ln = jnp.min(jnp.where(chunk == mv, lane, 16))
                gi = ch * 16 + ln
                if j < 16:
                    vlo = jnp.where(lane == j, mv, vlo)
                    ilo = jnp.where(lane == j, gi, ilo)
                else:
                    vhi = jnp.where(lane == j - 16, mv, vhi)
                    ihi = jnp.where(lane == j - 16, gi, ihi)
                newchunk = jnp.where(lane == ln, -1.0, chunk)
                plsc.store_scatter(p_v, [cidx], newchunk)
                plsc.store_scatter(cmax_v, [ch + zi], jnp.max(newchunk) + zf,
                                   mask=lane == 0)

            vals_v[pl.ds(0, 16)] = vlo
            vals_v[pl.ds(16, 16)] = vhi
            lidx_v[pl.ds(0, 16)] = ilo
            lidx_v[pl.ds(16, 16)] = ihi
            gidx_v[pl.ds(0, 16)] = ilo + b * S
            gidx_v[pl.ds(16, 16)] = ihi + b * S
            pltpu.sync_copy(vals_v, tvals_hbm.at[b])
            pltpu.sync_copy(lidx_v, tidx_hbm.at[b])
            pltpu.async_copy(x_hbm.at[gidx_v], rows_v, sem).wait()
            pltpu.sync_copy(rows_v, xg_hbm.at[pl.ds(b * _K, _K)])

    return sc_kernel(probs, x2d)


def _pn_body(gidx_ref, x_ref, w1_ref, b1_ref, w2_ref, b2_ref, idw_ref,
             idb_ref, gw_ref, gb_ref, tv_ref, ti_ref,
             tim_ref, ids_ref, g_ref, xg_ref, acc_ref, sem, *, nsteps):
    j = pl.program_id(0)
    R = xg_ref.shape[0]

    @pl.when(j == 0)
    def _():
        for r in range(R):
            pltpu.make_async_copy(x_ref.at[pl.ds(gidx_ref[r], 1)],
                                  xg_ref.at[pl.ds(r, 1)], sem).start()
        for r in range(R):
            pltpu.make_async_copy(x_ref.at[pl.ds(gidx_ref[r], 1)],
                                  xg_ref.at[pl.ds(r, 1)], sem).wait()

    f1 = _lrelu(jnp.dot(xg_ref[...], w1_ref[...],
                        preferred_element_type=jnp.float32) + b1_ref[...])
    part = jnp.dot(f1, w2_ref[...], preferred_element_type=jnp.float32)

    @pl.when(j == 0)
    def _():
        acc_ref[...] = part

    @pl.when(j > 0)
    def _():
        acc_ref[...] += part

    @pl.when(j == nsteps - 1)
    def _():
        N = idw_ref.shape[1]
        f2 = _lrelu(acc_ref[...] + b2_ref[...])
        logits = jnp.dot(f2, idw_ref[...],
                         preferred_element_type=jnp.float32) + idb_ref[...]
        m = jnp.max(logits, axis=1, keepdims=True)
        ncol = jax.lax.broadcasted_iota(jnp.int32, (R, N), 1)
        amax = jnp.min(jnp.where(logits == m, ncol, N), axis=1, keepdims=True)
        gl = jnp.sum(f2 * gw_ref[...], axis=1, keepdims=True) + gb_ref[...]
        gains = jax.nn.sigmoid(gl)
        mask = tv_ref[...] > 0.5
        tim_ref[...] = jnp.where(mask, ti_ref[...].astype(jnp.float32) * 0.01,
                                 0.0)
        ids_ref[...] = jnp.where(mask, amax, 0)
        g_ref[...] = jnp.where(mask, gains, 0.0)


def _param_net(x2d, gidx, W1, b1, W2, b2, idW, idb, gW, gb, tvals, tidx, cb):
    H = x2d.shape[1]
    R = gidx.shape[0]
    N = idW.shape[1]
    nsteps = H // cb
    body = functools.partial(_pn_body, nsteps=nsteps)
    grid_spec = pltpu.PrefetchScalarGridSpec(
        num_scalar_prefetch=1,
        grid=(nsteps,),
        in_specs=[
            pl.BlockSpec(memory_space=pl.ANY),
            pl.BlockSpec((H, cb), lambda j, gi: (0, j)),
            pl.BlockSpec((1, cb), lambda j, gi: (0, j)),
            pl.BlockSpec((cb, H), lambda j, gi: (j, 0)),
            pl.BlockSpec((1, H), lambda j, gi: (0, 0)),
            pl.BlockSpec((H, N), lambda j, gi: (0, 0)),
            pl.BlockSpec((1, N), lambda j, gi: (0, 0)),
            pl.BlockSpec((1, H), lambda j, gi: (0, 0)),
            pl.BlockSpec((1, 1), lambda j, gi: (0, 0)),
            pl.BlockSpec((R, 1), lambda j, gi: (0, 0)),
            pl.BlockSpec((R, 1), lambda j, gi: (0, 0)),
        ],
        out_specs=(
            pl.BlockSpec((R, 1), lambda j, gi: (0, 0)),
            pl.BlockSpec((R, 1), lambda j, gi: (0, 0)),
            pl.BlockSpec((R, 1), lambda j, gi: (0, 0)),
        ),
        scratch_shapes=[
            pltpu.VMEM((R, H), jnp.float32),
            pltpu.VMEM((R, H), jnp.float32),
            pltpu.SemaphoreType.DMA,
        ],
    )
    return pl.pallas_call(
        body,
        grid_spec=grid_spec,
        out_shape=(
            jax.ShapeDtypeStruct((R, 1), jnp.float32),
            jax.ShapeDtypeStruct((R, 1), jnp.int32),
            jax.ShapeDtypeStruct((R, 1), jnp.float32),
        ),
    )(gidx, x2d, W1, b1.reshape(1, H), W2, b2.reshape(1, H), idW,
      idb.reshape(1, N), gW.reshape(1, H), gb.reshape(1, 1), tvals, tidx)


def _pnx_body(xg_ref, w1_ref, b1_ref, w2_ref, b2_ref, idw_ref, idb_ref,
              gw_ref, gb_ref, tv_ref, ti_ref,
              tim_ref, ids_ref, g_ref, acc_ref, *, nsteps):
    j = pl.program_id(0)
    R = xg_ref.shape[0]
    f1 = _lrelu(jnp.dot(xg_ref[...], w1_ref[...],
                        preferred_element_type=jnp.float32) + b1_ref[...])
    part = jnp.dot(f1, w2_ref[...], preferred_element_type=jnp.float32)

    @pl.when(j == 0)
    def _():
        acc_ref[...] = part

    @pl.when(j > 0)
    def _():
        acc_ref[...] += part

    @pl.when(j == nsteps - 1)
    def _():
        N = idw_ref.shape[1]
        f2 = _lrelu(acc_ref[...] + b2_ref[...])
        logits = jnp.dot(f2, idw_ref[...],
                         preferred_element_type=jnp.float32) + idb_ref[...]
        m = jnp.max(logits, axis=1, keepdims=True)
        ncol = jax.lax.broadcasted_iota(jnp.int32, (R, N), 1)
        amax = jnp.min(jnp.where(logits == m, ncol, N), axis=1, keepdims=True)
        gl = jnp.sum(f2 * gw_ref[...], axis=1, keepdims=True) + gb_ref[...]
        gains = jax.nn.sigmoid(gl)
        mask = tv_ref[...] > 0.5
        tim_ref[...] = jnp.where(mask, ti_ref[...].astype(jnp.float32) * 0.01,
                                 0.0)
        ids_ref[...] = jnp.where(mask, amax, 0)
        g_ref[...] = jnp.where(mask, gains, 0.0)


def _param_net_xg(xg, W1, b1, W2, b2, idW, idb, gW, gb, tvals, tidx, cb):
    R, H = xg.shape
    N = idW.shape[1]
    nsteps = H // cb
    body = functools.partial(_pnx_body, nsteps=nsteps)
    return pl.pallas_call(
        body,
        grid=(nsteps,),
        in_specs=[
            pl.BlockSpec((R, H), lambda j: (0, 0)),
            pl.BlockSpec((H, cb), lambda j: (0, j)),
            pl.BlockSpec((1, cb), lambda j: (0, j)),
            pl.BlockSpec((cb, H), lambda j: (j, 0)),
            pl.BlockSpec((1, H), lambda j: (0, 0)),
            pl.BlockSpec((H, N), lambda j: (0, 0)),
            pl.BlockSpec((1, N), lambda j: (0, 0)),
            pl.BlockSpec((1, H), lambda j: (0, 0)),
            pl.BlockSpec((1, 1), lambda j: (0, 0)),
            pl.BlockSpec((R, 1), lambda j: (0, 0)),
            pl.BlockSpec((R, 1), lambda j: (0, 0)),
        ],
        out_specs=(
            pl.BlockSpec((R, 1), lambda j: (0, 0)),
            pl.BlockSpec((R, 1), lambda j: (0, 0)),
            pl.BlockSpec((R, 1), lambda j: (0, 0)),
        ),
        out_shape=(
            jax.ShapeDtypeStruct((R, 1), jnp.float32),
            jax.ShapeDtypeStruct((R, 1), jnp.int32),
            jax.ShapeDtypeStruct((R, 1), jnp.float32),
        ),
        scratch_shapes=[pltpu.VMEM((R, H), jnp.float32)],
    )(xg, W1, b1.reshape(1, H), W2, b2.reshape(1, H), idW,
      idb.reshape(1, N), gW.reshape(1, H), gb.reshape(1, 1), tvals, tidx)


def kernel(x, det_W1, det_b1, det_W2, det_b2, pn_W1, pn_b1, pn_W2, pn_b2,
           id_W, id_b, g_W, g_b):
    B, S, H = x.shape
    x2d = x.reshape(B * S, H)

    probs = _detector(x2d, det_W1, det_b1, det_W2, det_b2, rb=256)
    tvals, tidx, xg = _sc_topk_gather(probs.reshape(B, S), x2d)
    tim, ids, gains = _param_net_xg(
        xg, pn_W1, pn_b1, pn_W2, pn_b2, id_W, id_b,
        g_W, g_b, tvals.reshape(B * _K, 1), tidx.reshape(B * _K, 1), cb=512)
    return (tim.reshape(B, _K), ids.reshape(B, _K), gains.reshape(B, _K))


# speedup vs baseline: 1.0019x; 1.0019x over previous
import functools

import jax
import jax.numpy as jnp
from jax.experimental import pallas as pl
from jax.experimental.pallas import tpu as pltpu

_K = 32


def _lrelu(t):
    return jnp.where(t >= 0, t, 0.1 * t)


def _det_body(x_ref, w1_ref, b1_ref, w2_ref, b2_ref, o_ref):
    h = _lrelu(jnp.dot(x_ref[...], w1_ref[...],
                       preferred_element_type=jnp.float32) + b1_ref[...])
    logit = jnp.dot(h, w2_ref[...], preferred_element_type=jnp.float32)
    o_ref[...] = jax.nn.sigmoid(logit + b2_ref[...])


def _detector(x2d, W1, b1, W2, b2, rb):
    M, H = x2d.shape
    return pl.pallas_call(
        _det_body,
        grid=(M // rb,),
        in_specs=[
            pl.BlockSpec((rb, H), lambda i: (i, 0)),
            pl.BlockSpec((H, H), lambda i: (0, 0)),
            pl.BlockSpec((1, H), lambda i: (0, 0)),
            pl.BlockSpec((H, 1), lambda i: (0, 0)),
            pl.BlockSpec((1, 1), lambda i: (0, 0)),
        ],
        out_specs=pl.BlockSpec((rb, 1), lambda i: (i, 0)),
        out_shape=jax.ShapeDtypeStruct((M, 1), jnp.float32),
    )(x2d, W1, b1.reshape(1, H), W2, b2.reshape(1, 1))


def _topk_body(p_ref, vals_ref, idx_ref, gidx_ref):
    B, S = p_ref.shape
    p0 = p_ref[...]
    col = jax.lax.broadcasted_iota(jnp.int32, (B, S), 1)
    kcol = jax.lax.broadcasted_iota(jnp.int32, (B, _K), 1)

    def body(j, carry):
        p, vals, idxs = carry
        m = jnp.max(p, axis=1, keepdims=True)
        cand = jnp.where(p == m, col, S)
        i = jnp.min(cand, axis=1, keepdims=True)
        vals = jnp.where(kcol == j, m, vals)
        idxs = jnp.where(kcol == j, i, idxs)
        p = jnp.where(col == i, -1.0, p)
        return p, vals, idxs

    _, vals, idxs = jax.lax.fori_loop(
        0, _K, body,
        (p0, jnp.zeros((B, _K), jnp.float32), jnp.zeros((B, _K), jnp.int32)))
    vals_ref[...] = vals
    idx_ref[...] = idxs
    row = jax.lax.broadcasted_iota(jnp.int32, (B, _K), 0)
    gidx_ref[...] = idxs + row * S


def _topk(probs):
    B, S = probs.shape
    return pl.pallas_call(
        _topk_body,
        out_shape=(
            jax.ShapeDtypeStruct((B, _K), jnp.float32),
            jax.ShapeDtypeStruct((B, _K), jnp.int32),
            jax.ShapeDtypeStruct((B, _K), jnp.int32),
        ),
    )(probs)


from jax import lax
from jax.experimental.pallas import tpu_sc as plsc


def _sc_topk_gather(probs, x2d):
    B, S = probs.shape
    M, H = x2d.shape
    NCH = S // 16
    NG = NCH // 16
    mesh = plsc.VectorSubcoreMesh(core_axis_name="c", subcore_axis_name="s")

    @functools.partial(
        pl.kernel,
        mesh=mesh,
        compiler_params=pltpu.CompilerParams(needs_layout_passes=False),
        out_type=(
            jax.ShapeDtypeStruct((B, _K), jnp.float32),
            jax.ShapeDtypeStruct((B, _K), jnp.int32),
            jax.ShapeDtypeStruct((B * _K, H), jnp.float32),
        ),
        scratch_types=[
            pltpu.VMEM((S,), jnp.float32),
            pltpu.VMEM((NCH,), jnp.float32),
            pltpu.VMEM((_K,), jnp.float32),
            pltpu.VMEM((_K,), jnp.int32),
            pltpu.VMEM((_K,), jnp.int32),
            pltpu.VMEM((_K, H), jnp.float32),
            pltpu.SemaphoreType.DMA,
        ],
    )
    def sc_kernel(probs_hbm, x_hbm, tvals_hbm, tidx_hbm, xg_hbm,
                  p_v, cmax_v, vals_v, lidx_v, gidx_v, rows_v, sem):
        cid = lax.axis_index("c")
        sid = lax.axis_index("s")
        w = sid * 2 + cid

        @pl.when(w < B)
        def _():
            b = w
            lane = lax.broadcasted_iota(jnp.int32, (16,), 0)
            zi = jnp.zeros((16,), jnp.int32)
            zf = jnp.zeros((16,), jnp.float32)
            pltpu.sync_copy(probs_hbm.at[b], p_v)

            cm = []
            for g in range(NG):
                base = g * 256 + lane * 16
                acc = plsc.load_gather(p_v, [base])
                for k in range(1, 16):
                    acc = jnp.maximum(acc, plsc.load_gather(p_v, [base + k]))
                cm.append(acc)

            vlo = vhi = zf
            ilo = ihi = zi
            for j in range(_K):
                bestv = zf - 2.0
                bestc = zi
                for g in range(NG):
                    better = cm[g] > bestv
                    bestv = jnp.where(better, cm[g], bestv)
                    bestc = jnp.where(better, g * 16 + lane, bestc)
                mv = jnp.max(bestv)
                ch = jnp.min(jnp.where(bestv == mv, bestc, NCH))
                cidx = ch * 16 + lane
                chunk = plsc.load_gather(p_v, [cidx])
                ln = jnp.min(jnp.where(chunk == mv, lane, 16))
                gi = ch * 16 + ln
                if j < 16:
                    vlo = jnp.where(lane == j, mv, vlo)
                    ilo = jnp.where(lane == j, gi, ilo)
                else:
                    vhi = jnp.where(lane == j - 16, mv, vhi)
                    ihi = jnp.where(lane == j - 16, gi, ihi)
                newchunk = jnp.where(lane == ln, -1.0, chunk)
                plsc.store_scatter(p_v, [cidx], newchunk)
                ncm = jnp.max(newchunk)
                cdiv = ch // 16
                cmod = ch % 16
                for g in range(NG):
                    upd = jnp.logical_and(cdiv == g, lane == cmod)
                    cm[g] = jnp.where(upd, ncm, cm[g])

            vals_v[pl.ds(0, 16)] = vlo
            vals_v[pl.ds(16, 16)] = vhi
            lidx_v[pl.ds(0, 16)] = ilo
            lidx_v[pl.ds(16, 16)] = ihi
            gidx_v[pl.ds(0, 16)] = ilo + b * S
            gidx_v[pl.ds(16, 16)] = ihi + b * S
            pltpu.sync_copy(vals_v, tvals_hbm.at[b])
            pltpu.sync_copy(lidx_v, tidx_hbm.at[b])
            pltpu.async_copy(x_hbm.at[gidx_v], rows_v, sem).wait()
            pltpu.sync_copy(rows_v, xg_hbm.at[pl.ds(b * _K, _K)])

    return sc_kernel(probs, x2d)


def _pn_body(gidx_ref, x_ref, w1_ref, b1_ref, w2_ref, b2_ref, idw_ref,
             idb_ref, gw_ref, gb_ref, tv_ref, ti_ref,
             tim_ref, ids_ref, g_ref, xg_ref, acc_ref, sem, *, nsteps):
    j = pl.program_id(0)
    R = xg_ref.shape[0]

    @pl.when(j == 0)
    def _():
        for r in range(R):
            pltpu.make_async_copy(x_ref.at[pl.ds(gidx_ref[r], 1)],
                                  xg_ref.at[pl.ds(r, 1)], sem).start()
        for r in range(R):
            pltpu.make_async_copy(x_ref.at[pl.ds(gidx_ref[r], 1)],
                                  xg_ref.at[pl.ds(r, 1)], sem).wait()

    f1 = _lrelu(jnp.dot(xg_ref[...], w1_ref[...],
                        preferred_element_type=jnp.float32) + b1_ref[...])
    part = jnp.dot(f1, w2_ref[...], preferred_element_type=jnp.float32)

    @pl.when(j == 0)
    def _():
        acc_ref[...] = part

    @pl.when(j > 0)
    def _():
        acc_ref[...] += part

    @pl.when(j == nsteps - 1)
    def _():
        N = idw_ref.shape[1]
        f2 = _lrelu(acc_ref[...] + b2_ref[...])
        logits = jnp.dot(f2, idw_ref[...],
                         preferred_element_type=jnp.float32) + idb_ref[...]
        m = jnp.max(logits, axis=1, keepdims=True)
        ncol = jax.lax.broadcasted_iota(jnp.int32, (R, N), 1)
        amax = jnp.min(jnp.where(logits == m, ncol, N), axis=1, keepdims=True)
        gl = jnp.sum(f2 * gw_ref[...], axis=1, keepdims=True) + gb_ref[...]
        gains = jax.nn.sigmoid(gl)
        mask = tv_ref[...] > 0.5
        tim_ref[...] = jnp.where(mask, ti_ref[...].astype(jnp.float32) * 0.01,
                                 0.0)
        ids_ref[...] = jnp.where(mask, amax, 0)
        g_ref[...] = jnp.where(mask, gains, 0.0)


def _param_net(x2d, gidx, W1, b1, W2, b2, idW, idb, gW, gb, tvals, tidx, cb):
    H = x2d.shape[1]
    R = gidx.shape[0]
    N = idW.shape[1]
    nsteps = H // cb
    body = functools.partial(_pn_body, nsteps=nsteps)
    grid_spec = pltpu.PrefetchScalarGridSpec(
        num_scalar_prefetch=1,
        grid=(nsteps,),
        in_specs=[
            pl.BlockSpec(memory_space=pl.ANY),
            pl.BlockSpec((H, cb), lambda j, gi: (0, j)),
            pl.BlockSpec((1, cb), lambda j, gi: (0, j)),
            pl.BlockSpec((cb, H), lambda j, gi: (j, 0)),
            pl.BlockSpec((1, H), lambda j, gi: (0, 0)),
            pl.BlockSpec((H, N), lambda j, gi: (0, 0)),
            pl.BlockSpec((1, N), lambda j, gi: (0, 0)),
            pl.BlockSpec((1, H), lambda j, gi: (0, 0)),
            pl.BlockSpec((1, 1), lambda j, gi: (0, 0)),
            pl.BlockSpec((R, 1), lambda j, gi: (0, 0)),
            pl.BlockSpec((R, 1), lambda j, gi: (0, 0)),
        ],
        out_specs=(
            pl.BlockSpec((R, 1), lambda j, gi: (0, 0)),
            pl.BlockSpec((R, 1), lambda j, gi: (0, 0)),
            pl.BlockSpec((R, 1), lambda j, gi: (0, 0)),
        ),
        scratch_shapes=[
            pltpu.VMEM((R, H), jnp.float32),
            pltpu.VMEM((R, H), jnp.float32),
            pltpu.SemaphoreType.DMA,
        ],
    )
    return pl.pallas_call(
        body,
        grid_spec=grid_spec,
        out_shape=(
            jax.ShapeDtypeStruct((R, 1), jnp.float32),
            jax.ShapeDtypeStruct((R, 1), jnp.int32),
            jax.ShapeDtypeStruct((R, 1), jnp.float32),
        ),
    )(gidx, x2d, W1, b1.reshape(1, H), W2, b2.reshape(1, H), idW,
      idb.reshape(1, N), gW.reshape(1, H), gb.reshape(1, 1), tvals, tidx)


def _pnx_body(xg_ref, w1_ref, b1_ref, w2_ref, b2_ref, idw_ref, idb_ref,
              gw_ref, gb_ref, tv_ref, ti_ref,
              tim_ref, ids_ref, g_ref, acc_ref, *, nsteps):
    j = pl.program_id(0)
    R = xg_ref.shape[0]
    f1 = _lrelu(jnp.dot(xg_ref[...], w1_ref[...],
                        preferred_element_type=jnp.float32) + b1_ref[...])
    part = jnp.dot(f1, w2_ref[...], preferred_element_type=jnp.float32)

    @pl.when(j == 0)
    def _():
        acc_ref[...] = part

    @pl.when(j > 0)
    def _():
        acc_ref[...] += part

    @pl.when(j == nsteps - 1)
    def _():
        N = idw_ref.shape[1]
        f2 = _lrelu(acc_ref[...] + b2_ref[...])
        logits = jnp.dot(f2, idw_ref[...],
                         preferred_element_type=jnp.float32) + idb_ref[...]
        m = jnp.max(logits, axis=1, keepdims=True)
        ncol = jax.lax.broadcasted_iota(jnp.int32, (R, N), 1)
        amax = jnp.min(jnp.where(logits == m, ncol, N), axis=1, keepdims=True)
        gl = jnp.sum(f2 * gw_ref[...], axis=1, keepdims=True) + gb_ref[...]
        gains = jax.nn.sigmoid(gl)
        mask = tv_ref[...] > 0.5
        tim_ref[...] = jnp.where(mask, ti_ref[...].astype(jnp.float32) * 0.01,
                                 0.0)
        ids_ref[...] = jnp.where(mask, amax, 0)
        g_ref[...] = jnp.where(mask, gains, 0.0)


def _param_net_xg(xg, W1, b1, W2, b2, idW, idb, gW, gb, tvals, tidx, cb):
    R, H = xg.shape
    N = idW.shape[1]
    nsteps = H // cb
    body = functools.partial(_pnx_body, nsteps=nsteps)
    return pl.pallas_call(
        body,
        grid=(nsteps,),
        in_specs=[
            pl.BlockSpec((R, H), lambda j: (0, 0)),
            pl.BlockSpec((H, cb), lambda j: (0, j)),
            pl.BlockSpec((1, cb), lambda j: (0, j)),
            pl.BlockSpec((cb, H), lambda j: (j, 0)),
            pl.BlockSpec((1, H), lambda j: (0, 0)),
            pl.BlockSpec((H, N), lambda j: (0, 0)),
            pl.BlockSpec((1, N), lambda j: (0, 0)),
            pl.BlockSpec((1, H), lambda j: (0, 0)),
            pl.BlockSpec((1, 1), lambda j: (0, 0)),
            pl.BlockSpec((R, 1), lambda j: (0, 0)),
            pl.BlockSpec((R, 1), lambda j: (0, 0)),
        ],
        out_specs=(
            pl.BlockSpec((R, 1), lambda j: (0, 0)),
            pl.BlockSpec((R, 1), lambda j: (0, 0)),
            pl.BlockSpec((R, 1), lambda j: (0, 0)),
        ),
        out_shape=(
            jax.ShapeDtypeStruct((R, 1), jnp.float32),
            jax.ShapeDtypeStruct((R, 1), jnp.int32),
            jax.ShapeDtypeStruct((R, 1), jnp.float32),
        ),
        scratch_shapes=[pltpu.VMEM((R, H), jnp.float32)],
    )(xg, W1, b1.reshape(1, H), W2, b2.reshape(1, H), idW,
      idb.reshape(1, N), gW.reshape(1, H), gb.reshape(1, 1), tvals, tidx)


def kernel(x, det_W1, det_b1, det_W2, det_b2, pn_W1, pn_b1, pn_W2, pn_b2,
           id_W, id_b, g_W, g_b):
    B, S, H = x.shape
    x2d = x.reshape(B * S, H)

    probs = _detector(x2d, det_W1, det_b1, det_W2, det_b2, rb=256)
    tvals, tidx, xg = _sc_topk_gather(probs.reshape(B, S), x2d)
    tim, ids, gains = _param_net_xg(
        xg, pn_W1, pn_b1, pn_W2, pn_b2, id_W, id_b,
        g_W, g_b, tvals.reshape(B * _K, 1), tidx.reshape(B * _K, 1), cb=512)
    return (tim.reshape(B, _K), ids.reshape(B, _K), gains.reshape(B, _K))


# speedup vs baseline: 1.0019x; 1.0000x over previous
import functools

import jax
import jax.numpy as jnp
from jax.experimental import pallas as pl
from jax.experimental.pallas import tpu as pltpu

_K = 32


def _lrelu(t):
    return jnp.where(t >= 0, t, 0.1 * t)


def _det_body(x_ref, w1_ref, b1_ref, w2_ref, b2_ref, o_ref):
    h = _lrelu(jnp.dot(x_ref[...], w1_ref[...],
                       preferred_element_type=jnp.float32) + b1_ref[...])
    logit = jnp.dot(h, w2_ref[...], preferred_element_type=jnp.float32)
    o_ref[...] = jax.nn.sigmoid(logit + b2_ref[...])


def _detector(x2d, W1, b1, W2, b2, rb):
    M, H = x2d.shape
    return pl.pallas_call(
        _det_body,
        grid=(M // rb,),
        in_specs=[
            pl.BlockSpec((rb, H), lambda i: (i, 0)),
            pl.BlockSpec((H, H), lambda i: (0, 0)),
            pl.BlockSpec((1, H), lambda i: (0, 0)),
            pl.BlockSpec((H, 1), lambda i: (0, 0)),
            pl.BlockSpec((1, 1), lambda i: (0, 0)),
        ],
        out_specs=pl.BlockSpec((rb, 1), lambda i: (i, 0)),
        out_shape=jax.ShapeDtypeStruct((M, 1), jnp.float32),
    )(x2d, W1, b1.reshape(1, H), W2, b2.reshape(1, 1))


from jax import lax
from jax.experimental.pallas import tpu_sc as plsc


def _sc_topk_gather(probs, x2d):
    B, S = probs.shape
    M, H = x2d.shape
    NCH = S // 16
    NG = NCH // 16
    mesh = plsc.VectorSubcoreMesh(core_axis_name="c", subcore_axis_name="s")

    @functools.partial(
        pl.kernel,
        mesh=mesh,
        compiler_params=pltpu.CompilerParams(needs_layout_passes=False),
        out_type=(
            jax.ShapeDtypeStruct((B, _K), jnp.float32),
            jax.ShapeDtypeStruct((B, _K), jnp.int32),
            jax.ShapeDtypeStruct((B * _K, H), jnp.float32),
        ),
        scratch_types=[
            pltpu.VMEM((S,), jnp.float32),
            pltpu.VMEM((_K,), jnp.float32),
            pltpu.VMEM((_K,), jnp.int32),
            pltpu.VMEM((_K,), jnp.int32),
            pltpu.VMEM((_K, H), jnp.float32),
            pltpu.SemaphoreType.DMA,
        ],
    )
    def sc_kernel(probs_hbm, x_hbm, tvals_hbm, tidx_hbm, xg_hbm,
                  p_v, vals_v, lidx_v, gidx_v, rows_v, sem):
        cid = lax.axis_index("c")
        sid = lax.axis_index("s")
        w = sid * 2 + cid

        @pl.when(w < B)
        def _():
            b = w
            lane = lax.broadcasted_iota(jnp.int32, (16,), 0)
            zi = jnp.zeros((16,), jnp.int32)
            zf = jnp.zeros((16,), jnp.float32)
            pltpu.sync_copy(probs_hbm.at[b], p_v)

            cm = []
            for g in range(NG):
                base = g * 256 + lane * 16
                acc = plsc.load_gather(p_v, [base])
                for k in range(1, 16):
                    acc = jnp.maximum(acc, plsc.load_gather(p_v, [base + k]))
                cm.append(acc)

            vlo = vhi = zf
            ilo = ihi = zi
            for j in range(_K):
                bestv = zf - 2.0
                bestc = zi
                for g in range(NG):
                    better = cm[g] > bestv
                    bestv = jnp.where(better, cm[g], bestv)
                    bestc = jnp.where(better, g * 16 + lane, bestc)
                mv = jnp.max(bestv)
                ch = jnp.min(jnp.where(bestv == mv, bestc, NCH))
                cidx = ch * 16 + lane
                chunk = plsc.load_gather(p_v, [cidx])
                ln = jnp.min(jnp.where(chunk == mv, lane, 16))
                gi = ch * 16 + ln
                if j < 16:
                    vlo = jnp.where(lane == j, mv, vlo)
                    ilo = jnp.where(lane == j, gi, ilo)
                else:
                    vhi = jnp.where(lane == j - 16, mv, vhi)
                    ihi = jnp.where(lane == j - 16, gi, ihi)
                newchunk = jnp.where(lane == ln, -1.0, chunk)
                plsc.store_scatter(p_v, [cidx], newchunk)
                ncm = jnp.max(newchunk)
                cdiv = ch // 16
                cmod = ch % 16
                for g in range(NG):
                    upd = jnp.logical_and(cdiv == g, lane == cmod)
                    cm[g] = jnp.where(upd, ncm, cm[g])

            vals_v[pl.ds(0, 16)] = vlo
            vals_v[pl.ds(16, 16)] = vhi
            lidx_v[pl.ds(0, 16)] = ilo
            lidx_v[pl.ds(16, 16)] = ihi
            gidx_v[pl.ds(0, 16)] = ilo + b * S
            gidx_v[pl.ds(16, 16)] = ihi + b * S
            pltpu.sync_copy(vals_v, tvals_hbm.at[b])
            pltpu.sync_copy(lidx_v, tidx_hbm.at[b])
            pltpu.async_copy(x_hbm.at[gidx_v], rows_v, sem).wait()
            pltpu.sync_copy(rows_v, xg_hbm.at[pl.ds(b * _K, _K)])

    return sc_kernel(probs, x2d)


def _pnx_body(xg_ref, w1_ref, b1_ref, w2_ref, b2_ref, idw_ref, idb_ref,
              gw_ref, gb_ref, tv_ref, ti_ref,
              tim_ref, ids_ref, g_ref, acc_ref, *, nsteps):
    j = pl.program_id(0)
    R = xg_ref.shape[0]
    f1 = _lrelu(jnp.dot(xg_ref[...], w1_ref[...],
                        preferred_element_type=jnp.float32) + b1_ref[...])
    part = jnp.dot(f1, w2_ref[...], preferred_element_type=jnp.float32)

    @pl.when(j == 0)
    def _():
        acc_ref[...] = part

    @pl.when(j > 0)
    def _():
        acc_ref[...] += part

    @pl.when(j == nsteps - 1)
    def _():
        N = idw_ref.shape[1]
        f2 = _lrelu(acc_ref[...] + b2_ref[...])
        logits = jnp.dot(f2, idw_ref[...],
                         preferred_element_type=jnp.float32) + idb_ref[...]
        m = jnp.max(logits, axis=1, keepdims=True)
        ncol = jax.lax.broadcasted_iota(jnp.int32, (R, N), 1)
        amax = jnp.min(jnp.where(logits == m, ncol, N), axis=1, keepdims=True)
        gl = jnp.sum(f2 * gw_ref[...], axis=1, keepdims=True) + gb_ref[...]
        gains = jax.nn.sigmoid(gl)
        mask = tv_ref[...] > 0.5
        tim_ref[...] = jnp.where(mask, ti_ref[...].astype(jnp.float32) * 0.01,
                                 0.0)
        ids_ref[...] = jnp.where(mask, amax, 0)
        g_ref[...] = jnp.where(mask, gains, 0.0)


def _param_net_xg(xg, W1, b1, W2, b2, idW, idb, gW, gb, tvals, tidx, cb):
    R, H = xg.shape
    N = idW.shape[1]
    nsteps = H // cb
    body = functools.partial(_pnx_body, nsteps=nsteps)
    return pl.pallas_call(
        body,
        grid=(nsteps,),
        in_specs=[
            pl.BlockSpec((R, H), lambda j: (0, 0)),
            pl.BlockSpec((H, cb), lambda j: (0, j)),
            pl.BlockSpec((1, cb), lambda j: (0, j)),
            pl.BlockSpec((cb, H), lambda j: (j, 0)),
            pl.BlockSpec((1, H), lambda j: (0, 0)),
            pl.BlockSpec((H, N), lambda j: (0, 0)),
            pl.BlockSpec((1, N), lambda j: (0, 0)),
            pl.BlockSpec((1, H), lambda j: (0, 0)),
            pl.BlockSpec((1, 1), lambda j: (0, 0)),
            pl.BlockSpec((R, 1), lambda j: (0, 0)),
            pl.BlockSpec((R, 1), lambda j: (0, 0)),
        ],
        out_specs=(
            pl.BlockSpec((R, 1), lambda j: (0, 0)),
            pl.BlockSpec((R, 1), lambda j: (0, 0)),
            pl.BlockSpec((R, 1), lambda j: (0, 0)),
        ),
        out_shape=(
            jax.ShapeDtypeStruct((R, 1), jnp.float32),
            jax.ShapeDtypeStruct((R, 1), jnp.int32),
            jax.ShapeDtypeStruct((R, 1), jnp.float32),
        ),
        scratch_shapes=[pltpu.VMEM((R, H), jnp.float32)],
    )(xg, W1, b1.reshape(1, H), W2, b2.reshape(1, H), idW,
      idb.reshape(1, N), gW.reshape(1, H), gb.reshape(1, 1), tvals, tidx)


def kernel(x, det_W1, det_b1, det_W2, det_b2, pn_W1, pn_b1, pn_W2, pn_b2,
           id_W, id_b, g_W, g_b):
    B, S, H = x.shape
    x2d = x.reshape(B * S, H)

    probs = _detector(x2d, det_W1, det_b1, det_W2, det_b2, rb=256)
    tvals, tidx, xg = _sc_topk_gather(probs.reshape(B, S), x2d)
    tim, ids, gains = _param_net_xg(
        xg, pn_W1, pn_b1, pn_W2, pn_b2, id_W, id_b,
        g_W, g_b, tvals.reshape(B * _K, 1), tidx.reshape(B * _K, 1), cb=512)
    return (tim.reshape(B, _K), ids.reshape(B, _K), gains.reshape(B, _K))
